# baseline (speedup 1.0000x reference)
import jax
import jax.numpy as jnp
from jax.experimental import pallas as pl
from jax.experimental.pallas import tpu as pltpu

_BLOCK_H = 8


def _body(w_ref, x_ref, logits_ref, tgt_ref, out_ref, acc_ref):
    i = pl.program_id(0)

    @pl.when(i == 0)
    def _margin():
        lg = logits_ref[...]
        t = tgt_ref[...]
        col = jax.lax.broadcasted_iota(jnp.int32, lg.shape, 1)
        onehot = col == t
        masked = jnp.where(onehot, -jnp.inf, lg)
        max_other = jnp.max(masked, axis=1)
        true_score = jnp.sum(jnp.where(onehot, lg, 0.0), axis=1)
        margin = jnp.maximum(true_score - max_other, -10.0)
        out_ref[0, 1] = jnp.sum(margin)
        acc_ref[...] = jnp.zeros_like(acc_ref)

    wt = 127.5 * (jnp.tanh(w_ref[...]) + 1.0)
    d = wt - x_ref[...]
    acc_ref[...] += jnp.sum(d * d, axis=(0, 1))

    @pl.when(i == pl.num_programs(0) - 1)
    def _finish():
        out_ref[0, 0] = jnp.sum(acc_ref[...])


def kernel(w, x, logits, targets):
    b, ch, h, wd = w.shape
    batch, n_classes = logits.shape
    grid = h // _BLOCK_H

    out = pl.pallas_call(
        _body,
        grid=(grid,),
        in_specs=[
            pl.BlockSpec((b, ch, _BLOCK_H, wd), lambda i: (0, 0, i, 0)),
            pl.BlockSpec((b, ch, _BLOCK_H, wd), lambda i: (0, 0, i, 0)),
            pl.BlockSpec((batch, n_classes), lambda i: (0, 0)),
            pl.BlockSpec((batch, 1), lambda i: (0, 0)),
        ],
        out_specs=pl.BlockSpec(memory_space=pltpu.SMEM),
        out_shape=jax.ShapeDtypeStruct((1, 2), jnp.float32),
        scratch_shapes=[pltpu.VMEM((_BLOCK_H, wd), jnp.float32)],
        compiler_params=pltpu.CompilerParams(
            dimension_semantics=("arbitrary",),
        ),
    )(w, x, logits, targets)

    n_total = b * ch * h * wd
    return out[0, 0] / n_total + 0.5 * out[0, 1] / batch
